# speedup vs baseline: 4.8035x; 4.8035x over previous
import jax
import jax.numpy as jnp
from jax.experimental import pallas as pl
from jax.experimental.pallas import tpu as pltpu

T = 512
D = 2048
E = 64
TOP_K = 2
I = 512
NS = 2
N_GROUP = 8
GROUP_SIZE = E // N_GROUP
TOPK_GROUP = 4
CAP = 48
SCALE = 16.0


def _moe_kernel(hs_ref, gw_ref, wgu_ref, wd_ref, sgu_ref, sd_ref,
                out_ref, w_scr, pos_scr):
    e = pl.program_id(0)
    lane = jax.lax.broadcasted_iota(jnp.int32, (T, E), 1)

    @pl.when(e == 0)
    def _prologue():
        hs = hs_ref[:, :]
        logits = jnp.dot(hs, gw_ref[:, :], preferred_element_type=jnp.float32)
        mx = jnp.max(logits, axis=-1, keepdims=True)
        ex = jnp.exp(logits - mx)
        scores = ex / jnp.sum(ex, axis=-1, keepdims=True)
        lane_group = lane // GROUP_SIZE
        gsb = jnp.zeros((T, E), jnp.float32)
        for g in range(N_GROUP):
            gm = jnp.max(jnp.where(lane_group == g, scores, -1.0),
                         axis=-1, keepdims=True)
            gsb = jnp.where(lane_group == g, gm, gsb)
        sel = jnp.zeros((T, E), jnp.bool_)
        cur = gsb
        for _ in range(TOPK_GROUP):
            gmx = jnp.max(cur, axis=-1, keepdims=True)
            lidx = jnp.min(jnp.where(cur == gmx, lane, E),
                           axis=-1, keepdims=True)
            sgrp = lidx // GROUP_SIZE
            hit = lane_group == sgrp
            sel = jnp.logical_or(sel, hit)
            cur = jnp.where(hit, -1.0, cur)
        ms = jnp.where(sel, scores, 0.0)
        v1 = jnp.max(ms, axis=-1, keepdims=True)
        l1 = jnp.min(jnp.where(ms == v1, lane, E), axis=-1, keepdims=True)
        ms2 = jnp.where(lane == l1, -1.0, ms)
        v2 = jnp.max(ms2, axis=-1, keepdims=True)
        l2 = jnp.min(jnp.where(ms2 == v2, lane, E), axis=-1, keepdims=True)
        s = v1 + v2 + 1e-20
        wmat = (jnp.where(lane == l1, v1 / s, 0.0)
                + jnp.where(lane == l2, v2 / s, 0.0))
        w_scr[:, :] = wmat
        mmat = (wmat > 0.0).astype(jnp.float32)
        r_i = jax.lax.broadcasted_iota(jnp.int32, (T, T), 0)
        c_i = jax.lax.broadcasted_iota(jnp.int32, (T, T), 1)
        tril = (r_i >= c_i).astype(jnp.float32)
        pos_scr[:, :] = jnp.dot(tril, mmat,
                                preferred_element_type=jnp.float32) - 1.0
        sg = jnp.dot(hs, sgu_ref[:, :I * NS],
                     preferred_element_type=jnp.float32)
        su = jnp.dot(hs, sgu_ref[:, I * NS:],
                     preferred_element_type=jnp.float32)
        sh = sg * jax.nn.sigmoid(sg) * su
        out_ref[:, :] = jnp.dot(sh, sd_ref[:, :],
                                preferred_element_type=jnp.float32)

    w_col = jnp.sum(jnp.where(lane == e, w_scr[:, :], 0.0),
                    axis=-1, keepdims=True)
    pos_col = jnp.sum(jnp.where(lane == e, pos_scr[:, :], 0.0),
                      axis=-1, keepdims=True)
    lane_f = lane.astype(jnp.float32)
    p = jnp.where((pos_col == lane_f) & (w_col > 0.0) & (lane < CAP),
                  1.0, 0.0)
    xe = jax.lax.dot_general(p, hs_ref[:, :], (((0,), (0,)), ((), ())),
                             preferred_element_type=jnp.float32)
    gue = jnp.dot(xe, wgu_ref[0], preferred_element_type=jnp.float32)
    ge = gue[:, :I]
    ue = gue[:, I:]
    he = ge * jax.nn.sigmoid(ge) * ue
    ye = jnp.dot(he, wd_ref[0], preferred_element_type=jnp.float32)
    pw = p * (w_col * SCALE)
    out_ref[:, :] += jnp.dot(pw, ye, preferred_element_type=jnp.float32)


def kernel(hidden_states, gate_w, w_gate_up, w_down, shared_gate_up,
           shared_down):
    return pl.pallas_call(
        _moe_kernel,
        grid=(E,),
        in_specs=[
            pl.BlockSpec((T, D), lambda e: (0, 0)),
            pl.BlockSpec((D, E), lambda e: (0, 0)),
            pl.BlockSpec((1, D, 2 * I), lambda e: (e, 0, 0)),
            pl.BlockSpec((1, I, D), lambda e: (e, 0, 0)),
            pl.BlockSpec((D, 2 * I * NS), lambda e: (0, 0)),
            pl.BlockSpec((I * NS, D), lambda e: (0, 0)),
        ],
        out_specs=pl.BlockSpec((T, D), lambda e: (0, 0)),
        out_shape=jax.ShapeDtypeStruct((T, D), jnp.float32),
        scratch_shapes=[
            pltpu.VMEM((T, E), jnp.float32),
            pltpu.VMEM((T, E), jnp.float32),
        ],
        compiler_params=pltpu.CompilerParams(
            dimension_semantics=("arbitrary",),
            vmem_limit_bytes=120 * 1024 * 1024,
        ),
    )(hidden_states, gate_w, w_gate_up, w_down, shared_gate_up, shared_down)
